# baseline (speedup 1.0000x reference)
import functools

import jax
import jax.numpy as jnp
from jax import lax
from jax.experimental import pallas as pl
from jax.experimental.pallas import tpu as pltpu
from jax.experimental.pallas import tpu_sc as plsc

N_NODES_C = 10000
D_C = 256
NC = 2
NS = 16
NW = NC * NS
L = 16
NODES_T = 320
LAST_T = N_NODES_C - (NW - 1) * NODES_T
ACC_ROWS = NODES_T + 8
DUMMY = NODES_T
WINDOW = 1024
K = 16
NBUF = 8
CAP = 2048


def _sc_segment_sum(feature, src_i32, dst_i32, n_windows):
  mesh = plsc.VectorSubcoreMesh(
      core_axis_name="c", subcore_axis_name="s", num_cores=NC, num_subcores=NS)

  @functools.partial(
      pl.kernel,
      out_type=jax.ShapeDtypeStruct((N_NODES_C, D_C), jnp.float32),
      mesh=mesh,
      compiler_params=pltpu.CompilerParams(needs_layout_passes=False),
      scratch_types=[
          pltpu.VMEM((ACC_ROWS, D_C), jnp.float32),
          pltpu.VMEM((2, WINDOW), jnp.int32),
          pltpu.VMEM((2, WINDOW), jnp.int32),
          pltpu.VMEM((CAP + L,), jnp.int32),
          pltpu.VMEM((CAP + L,), jnp.int32),
          pltpu.VMEM((NBUF, K, D_C), jnp.float32),
          pltpu.VMEM((NBUF, K), jnp.int32),
          pltpu.SemaphoreType.DMA,
          pltpu.SemaphoreType.DMA,
      ],
  )
  def sc_kernel(feat_hbm, src_hbm, dst_hbm, out_hbm,
                acc, swin, dwin, ksrc, kdst, ring, cidx, sem, wsem):
    c = lax.axis_index("c")
    s = lax.axis_index("s")
    wid = s * NC + c
    base = wid * NODES_T

    zvec = jnp.zeros((L,), jnp.float32)

    def zero_row(r, _):
      for g in range(D_C // L):
        acc[r, pl.ds(g * L, L)] = zvec
      return 0
    lax.fori_loop(0, ACC_ROWS, zero_row, 0)

    basev = jnp.full((L,), base, jnp.int32)
    limv = jnp.full((L,), NODES_T, jnp.int32)
    zerov = jnp.zeros((L,), jnp.int32)
    dummyv = jnp.full((L,), DUMMY, jnp.int32)
    capm = jnp.full((L,), CAP - 1, jnp.int32)
    trash = jnp.full((L,), CAP, jnp.int32)

    def load_window(w):
      slot = lax.rem(w, jnp.int32(2))
      ebase = w * WINDOW
      pltpu.async_copy(src_hbm.at[pl.ds(ebase, WINDOW)], swin.at[slot], wsem)
      pltpu.async_copy(dst_hbm.at[pl.ds(ebase, WINDOW)], dwin.at[slot], wsem)

    def wait_window():
      pltpu.make_async_copy(
          src_hbm.at[pl.ds(0, WINDOW)], swin.at[0], wsem).wait()
      pltpu.make_async_copy(
          dst_hbm.at[pl.ds(0, WINDOW)], dwin.at[0], wsem).wait()

    def fire(f):
      slot = lax.rem(f, jnp.int32(NBUF))
      off = lax.bitwise_and(f * K, jnp.int32(CAP - 1))
      cidx[slot, pl.ds(0, L)] = ksrc[pl.ds(off, L)]
      pltpu.async_copy(feat_hbm.at[cidx.at[slot]], ring.at[slot], sem)

    def drain_one():
      pltpu.make_async_copy(
          feat_hbm.at[pl.ds(0, K)], ring.at[0], sem).wait()

    onev = jnp.full((L,), 1, jnp.int32)
    lanev = lax.iota(jnp.int32, L)

    def accumulate(f):
      slot = lax.rem(f, jnp.int32(NBUF))
      off = lax.bitwise_and(f * K, jnp.int32(CAP - 1))
      d = kdst[pl.ds(off, L)]
      slotv = jnp.full((L,), slot, jnp.int32)

      def colblk(cb, colv):
        for _ in range(16):
          x = plsc.load_gather(ring, [slotv, lanev, colv])
          plsc.addupdate_scatter(acc, [d, colv], x)
          colv = colv + onev
        return colv
      lax.fori_loop(0, D_C // 16, colblk, jnp.zeros((L,), jnp.int32))

    load_window(jnp.int32(0))

    def window_body(w, carry):
      wr, qf = carry
      wait_window()
      @pl.when(w + 1 < n_windows)
      def _():
        load_window(w + 1)
      wslot = lax.rem(w, jnp.int32(2))

      def filt(g, cnt):
        for sub in range(2):
          gg = g * 2 + sub
          d = dwin[wslot, pl.ds(gg * L, L)]
          sv = swin[wslot, pl.ds(gg * L, L)]
          dl = d - basev
          ok = (dl >= zerov) & (dl < limv)
          oki = ok.astype(jnp.int32)
          cum = plsc.cumsum(oki)
          pos = lax.bitwise_and(
              cum - oki + jnp.full((L,), cnt, jnp.int32), capm)
          pos = jnp.where(ok, pos, trash)
          plsc.store_scatter(kdst, [pos], dl)
          plsc.store_scatter(ksrc, [pos], sv)
          cnt = cnt + cum[L - 1]
        return cnt
      wr2 = lax.fori_loop(0, WINDOW // (2 * L), filt, wr)

      nf = lax.div(wr2, jnp.int32(K))

      def fire_body(f, _):
        @pl.when(f >= NBUF)
        def _():
          drain_one()
          accumulate(f - NBUF)
        fire(f)
        return 0
      lax.fori_loop(qf, nf, fire_body, 0)
      return (wr2, nf)

    wr, qf = lax.fori_loop(0, n_windows, window_body,
                           (jnp.int32(0), jnp.int32(0)))

    rem = wr - qf * K
    @pl.when(rem > 0)
    def _():
      off = lax.bitwise_and(qf * K, jnp.int32(CAP - 1))
      io = lax.iota(jnp.int32, L)
      m = io < jnp.full((L,), rem, jnp.int32)
      kdst[pl.ds(off, L)] = jnp.where(m, kdst[pl.ds(off, L)], dummyv)
      ksrc[pl.ds(off, L)] = jnp.where(m, ksrc[pl.ds(off, L)], zerov)
      @pl.when(qf >= NBUF)
      def _():
        drain_one()
        accumulate(qf - NBUF)
      fire(qf)
    nf_all = qf + (rem > 0).astype(jnp.int32)

    def drain_body(f, _):
      drain_one()
      accumulate(f)
      return 0
    lax.fori_loop(jnp.maximum(nf_all - NBUF, 0), nf_all, drain_body, 0)

    @pl.when(wid < NW - 1)
    def _():
      pltpu.sync_copy(acc.at[pl.ds(0, NODES_T)],
                      out_hbm.at[pl.ds(base, NODES_T)])
    @pl.when(wid == NW - 1)
    def _():
      pltpu.sync_copy(acc.at[pl.ds(0, LAST_T)],
                      out_hbm.at[pl.ds(base, LAST_T)])

  return sc_kernel(feature, src_i32, dst_i32)


def _tc_linear_body(h_ref, wt_ref, b_ref, out_ref):
  out_ref[...] = (
      jnp.dot(h_ref[...], wt_ref[...], preferred_element_type=jnp.float32)
      + b_ref[0:1, :])


def _tc_linear(h, wt, b2d):
  m_blk = 1000
  grid = (h.shape[0] // m_blk,)
  return pl.pallas_call(
      _tc_linear_body,
      grid=grid,
      in_specs=[
          pl.BlockSpec((m_blk, D_C), lambda i: (i, 0)),
          pl.BlockSpec((D_C, D_C), lambda i: (0, 0)),
          pl.BlockSpec((8, D_C), lambda i: (0, 0)),
      ],
      out_specs=pl.BlockSpec((m_blk, D_C), lambda i: (i, 0)),
      out_shape=jax.ShapeDtypeStruct((h.shape[0], D_C), jnp.float32),
  )(h, wt, b2d)


@jax.jit
def kernel(feature, edge_index, W, b):
  src = edge_index[0].astype(jnp.int32)
  dst = edge_index[1].astype(jnp.int32)
  n_edges = src.shape[0]
  n_windows = -(-n_edges // WINDOW)
  e_pad = n_windows * WINDOW
  if e_pad != n_edges:
    pad = e_pad - n_edges
    src = jnp.concatenate([src, jnp.zeros((pad,), jnp.int32)])
    dst = jnp.concatenate([dst, jnp.full((pad,), N_NODES_C, jnp.int32)])
  h = _sc_segment_sum(feature, src, dst, n_windows)
  return _tc_linear(h, W.T, jnp.tile(b.reshape(1, D_C), (8, 1)))


# speedup vs baseline: 3.2246x; 3.2246x over previous
import functools

import jax
import jax.numpy as jnp
from jax import lax
from jax.experimental import pallas as pl
from jax.experimental.pallas import tpu as pltpu
from jax.experimental.pallas import tpu_sc as plsc

N_NODES_C = 10000
D_C = 256
NC = 2
NS = 16
NW = NC * NS
L = 16
NODES_T = 320
LAST_T = N_NODES_C - (NW - 1) * NODES_T
ACC_ROWS = NODES_T + 8
DUMMY = NODES_T
WINDOW = 2048
K = 16
NBUF = 7
CAP = 4096


def _sc_segment_sum(feature, src_i32, dst_i32, n_windows):
  mesh = plsc.VectorSubcoreMesh(
      core_axis_name="c", subcore_axis_name="s", num_cores=NC, num_subcores=NS)

  @functools.partial(
      pl.kernel,
      out_type=jax.ShapeDtypeStruct((N_NODES_C, D_C), jnp.float32),
      mesh=mesh,
      compiler_params=pltpu.CompilerParams(needs_layout_passes=False),
      scratch_types=[
          pltpu.VMEM((ACC_ROWS, D_C), jnp.float32),
          pltpu.VMEM((2, WINDOW), jnp.int32),
          pltpu.VMEM((2, WINDOW), jnp.int32),
          pltpu.VMEM((CAP + L,), jnp.int32),
          pltpu.VMEM((CAP + L,), jnp.int32),
          pltpu.VMEM((NBUF, K, D_C), jnp.float32),
          pltpu.VMEM((NBUF, K), jnp.int32),
          pltpu.SemaphoreType.DMA,
          pltpu.SemaphoreType.DMA,
      ],
  )
  def sc_kernel(feat_hbm, src_hbm, dst_hbm, out_hbm,
                acc, swin, dwin, ksrc, kdst, ring, cidx, sem, wsem):
    c = lax.axis_index("c")
    s = lax.axis_index("s")
    wid = s * NC + c
    base = wid * NODES_T

    zvec = jnp.zeros((L,), jnp.float32)

    def zero_row(r, _):
      for g in range(D_C // L):
        acc[r, pl.ds(g * L, L)] = zvec
      return 0
    lax.fori_loop(0, ACC_ROWS, zero_row, 0)

    basev = jnp.full((L,), base, jnp.int32)
    limv = jnp.full((L,), NODES_T, jnp.int32)
    zerov = jnp.zeros((L,), jnp.int32)
    dummyv = jnp.full((L,), DUMMY, jnp.int32)
    capm = jnp.full((L,), CAP - 1, jnp.int32)
    trash = jnp.full((L,), CAP, jnp.int32)

    def load_window(w):
      slot = lax.rem(w, jnp.int32(2))
      ebase = w * WINDOW
      pltpu.async_copy(src_hbm.at[pl.ds(ebase, WINDOW)], swin.at[slot], wsem)
      pltpu.async_copy(dst_hbm.at[pl.ds(ebase, WINDOW)], dwin.at[slot], wsem)

    def wait_window():
      pltpu.make_async_copy(
          src_hbm.at[pl.ds(0, WINDOW)], swin.at[0], wsem).wait()
      pltpu.make_async_copy(
          dst_hbm.at[pl.ds(0, WINDOW)], dwin.at[0], wsem).wait()

    def fire(f):
      slot = lax.rem(f, jnp.int32(NBUF))
      off = lax.bitwise_and(f * K, jnp.int32(CAP - 1))
      cidx[slot, pl.ds(0, L)] = ksrc[pl.ds(off, L)]
      pltpu.async_copy(feat_hbm.at[cidx.at[slot]], ring.at[slot], sem)

    def drain_one():
      pltpu.make_async_copy(
          feat_hbm.at[pl.ds(0, K)], ring.at[0], sem).wait()

    def accumulate(f):
      slot = lax.rem(f, jnp.int32(NBUF))
      off = lax.bitwise_and(f * K, jnp.int32(CAP - 1))
      d = kdst[pl.ds(off, L)]
      for lane in range(L):
        row = d[lane]
        for grp in range(D_C // L):
          sl = pl.ds(grp * L, L)
          acc[row, sl] = acc[row, sl] + ring[slot, lane, sl]

    load_window(jnp.int32(0))

    def window_body(w, carry):
      wr, qf = carry
      wait_window()
      @pl.when(w + 1 < n_windows)
      def _():
        load_window(w + 1)
      wslot = lax.rem(w, jnp.int32(2))

      def filt(g, cnt):
        d = dwin[wslot, pl.ds(g * L, L)]
        sv = swin[wslot, pl.ds(g * L, L)]
        dl = d - basev
        ok = (dl >= zerov) & (dl < limv)
        oki = ok.astype(jnp.int32)
        cum = plsc.cumsum(oki)
        pos = lax.bitwise_and(cum - oki + jnp.full((L,), cnt, jnp.int32), capm)
        pos = jnp.where(ok, pos, trash)
        plsc.store_scatter(kdst, [pos], dl)
        plsc.store_scatter(ksrc, [pos], sv)
        return cnt + cum[L - 1]
      wr2 = lax.fori_loop(0, WINDOW // L, filt, wr)

      nf = lax.div(wr2, jnp.int32(K))

      def fire_body(f, _):
        @pl.when(f >= NBUF)
        def _():
          drain_one()
          accumulate(f - NBUF)
        fire(f)
        return 0
      lax.fori_loop(qf, nf, fire_body, 0)
      return (wr2, nf)

    wr, qf = lax.fori_loop(0, n_windows, window_body,
                           (jnp.int32(0), jnp.int32(0)))

    rem = wr - qf * K
    @pl.when(rem > 0)
    def _():
      off = lax.bitwise_and(qf * K, jnp.int32(CAP - 1))
      io = lax.iota(jnp.int32, L)
      m = io < jnp.full((L,), rem, jnp.int32)
      kdst[pl.ds(off, L)] = jnp.where(m, kdst[pl.ds(off, L)], dummyv)
      ksrc[pl.ds(off, L)] = jnp.where(m, ksrc[pl.ds(off, L)], zerov)
      @pl.when(qf >= NBUF)
      def _():
        drain_one()
        accumulate(qf - NBUF)
      fire(qf)
    nf_all = qf + (rem > 0).astype(jnp.int32)

    def drain_body(f, _):
      drain_one()
      accumulate(f)
      return 0
    lax.fori_loop(jnp.maximum(nf_all - NBUF, 0), nf_all, drain_body, 0)

    @pl.when(wid < NW - 1)
    def _():
      pltpu.sync_copy(acc.at[pl.ds(0, NODES_T)],
                      out_hbm.at[pl.ds(base, NODES_T)])
    @pl.when(wid == NW - 1)
    def _():
      pltpu.sync_copy(acc.at[pl.ds(0, LAST_T)],
                      out_hbm.at[pl.ds(base, LAST_T)])

  return sc_kernel(feature, src_i32, dst_i32)


def _tc_linear_body(h_ref, wt_ref, b_ref, out_ref):
  out_ref[...] = (
      jnp.dot(h_ref[...], wt_ref[...], preferred_element_type=jnp.float32)
      + b_ref[0:1, :])


def _tc_linear(h, wt, b2d):
  m_blk = 1000
  grid = (h.shape[0] // m_blk,)
  return pl.pallas_call(
      _tc_linear_body,
      grid=grid,
      in_specs=[
          pl.BlockSpec((m_blk, D_C), lambda i: (i, 0)),
          pl.BlockSpec((D_C, D_C), lambda i: (0, 0)),
          pl.BlockSpec((8, D_C), lambda i: (0, 0)),
      ],
      out_specs=pl.BlockSpec((m_blk, D_C), lambda i: (i, 0)),
      out_shape=jax.ShapeDtypeStruct((h.shape[0], D_C), jnp.float32),
  )(h, wt, b2d)


@jax.jit
def kernel(feature, edge_index, W, b):
  src = edge_index[0].astype(jnp.int32)
  dst = edge_index[1].astype(jnp.int32)
  n_edges = src.shape[0]
  n_windows = -(-n_edges // WINDOW)
  e_pad = n_windows * WINDOW
  if e_pad != n_edges:
    pad = e_pad - n_edges
    src = jnp.concatenate([src, jnp.zeros((pad,), jnp.int32)])
    dst = jnp.concatenate([dst, jnp.full((pad,), N_NODES_C, jnp.int32)])
  h = _sc_segment_sum(feature, src, dst, n_windows)
  return _tc_linear(h, W.T, jnp.tile(b.reshape(1, D_C), (8, 1)))


# speedup vs baseline: 3.8929x; 1.2072x over previous
import functools

import jax
import jax.numpy as jnp
from jax import lax
from jax.experimental import pallas as pl
from jax.experimental.pallas import tpu as pltpu
from jax.experimental.pallas import tpu_sc as plsc

N_NODES_C = 10000
D_C = 256
NC = 2
NS = 16
NW = NC * NS
L = 16
NODES_T = 320
LAST_T = N_NODES_C - (NW - 1) * NODES_T
ACC_ROWS = NODES_T + 8
DUMMY = NODES_T
WINDOW = 2048
K = 16
NBUF = 7
CAP = 4096


def _sc_segment_sum(feature, src_i32, dst_i32, n_windows):
  mesh = plsc.VectorSubcoreMesh(
      core_axis_name="c", subcore_axis_name="s", num_cores=NC, num_subcores=NS)

  @functools.partial(
      pl.kernel,
      out_type=jax.ShapeDtypeStruct((N_NODES_C, D_C), jnp.float32),
      mesh=mesh,
      compiler_params=pltpu.CompilerParams(needs_layout_passes=False),
      scratch_types=[
          pltpu.VMEM((ACC_ROWS, D_C), jnp.float32),
          pltpu.VMEM((2, WINDOW), jnp.int32),
          pltpu.VMEM((2, WINDOW), jnp.int32),
          pltpu.VMEM((CAP + L,), jnp.int32),
          pltpu.VMEM((CAP + L,), jnp.int32),
          pltpu.VMEM((NBUF, K, D_C), jnp.float32),
          pltpu.VMEM((NBUF, K), jnp.int32),
          pltpu.SemaphoreType.DMA,
          pltpu.SemaphoreType.DMA,
      ],
  )
  def sc_kernel(feat_hbm, src_hbm, dst_hbm, out_hbm,
                acc, swin, dwin, ksrc, kdst, ring, cidx, sem, wsem):
    c = lax.axis_index("c")
    s = lax.axis_index("s")
    wid = s * NC + c
    base = wid * NODES_T

    zvec = jnp.zeros((L,), jnp.float32)

    def zero_row(r, _):
      for g in range(D_C // L):
        acc[r, pl.ds(g * L, L)] = zvec
      return 0
    lax.fori_loop(0, ACC_ROWS, zero_row, 0)

    basev = jnp.full((L,), base, jnp.int32)
    limv = jnp.full((L,), NODES_T, jnp.int32)
    zerov = jnp.zeros((L,), jnp.int32)
    dummyv = jnp.full((L,), DUMMY, jnp.int32)
    capm = jnp.full((L,), CAP - 1, jnp.int32)
    trash = jnp.full((L,), CAP, jnp.int32)

    def load_window(w):
      slot = lax.rem(w, jnp.int32(2))
      ebase = w * WINDOW
      pltpu.async_copy(src_hbm.at[pl.ds(ebase, WINDOW)], swin.at[slot], wsem)
      pltpu.async_copy(dst_hbm.at[pl.ds(ebase, WINDOW)], dwin.at[slot], wsem)

    def wait_window():
      pltpu.make_async_copy(
          src_hbm.at[pl.ds(0, WINDOW)], swin.at[0], wsem).wait()
      pltpu.make_async_copy(
          dst_hbm.at[pl.ds(0, WINDOW)], dwin.at[0], wsem).wait()

    def fire(f):
      slot = lax.rem(f, jnp.int32(NBUF))
      off = lax.bitwise_and(f * K, jnp.int32(CAP - 1))
      cidx[slot, pl.ds(0, L)] = ksrc[pl.ds(off, L)]
      pltpu.async_copy(feat_hbm.at[cidx.at[slot]], ring.at[slot], sem)

    def drain_one():
      pltpu.make_async_copy(
          feat_hbm.at[pl.ds(0, K)], ring.at[0], sem).wait()

    def accumulate(f):
      slot = lax.rem(f, jnp.int32(NBUF))
      off = lax.bitwise_and(f * K, jnp.int32(CAP - 1))
      d = kdst[pl.ds(off, L)]
      for lane in range(L):
        row = d[lane]
        for grp in range(D_C // L):
          sl = pl.ds(grp * L, L)
          plsc.addupdate(acc.at[row, sl], ring[slot, lane, sl])

    load_window(jnp.int32(0))

    def window_body(w, carry):
      wr, qf = carry
      wait_window()
      @pl.when(w + 1 < n_windows)
      def _():
        load_window(w + 1)
      wslot = lax.rem(w, jnp.int32(2))

      def filt(g, cnt):
        d = dwin[wslot, pl.ds(g * L, L)]
        sv = swin[wslot, pl.ds(g * L, L)]
        dl = d - basev
        ok = (dl >= zerov) & (dl < limv)
        oki = ok.astype(jnp.int32)
        cum = plsc.cumsum(oki)
        pos = lax.bitwise_and(cum - oki + jnp.full((L,), cnt, jnp.int32), capm)
        pos = jnp.where(ok, pos, trash)
        plsc.store_scatter(kdst, [pos], dl)
        plsc.store_scatter(ksrc, [pos], sv)
        return cnt + cum[L - 1]
      wr2 = lax.fori_loop(0, WINDOW // L, filt, wr)

      nf = lax.div(wr2, jnp.int32(K))

      def fire_body(f, _):
        @pl.when(f >= NBUF)
        def _():
          drain_one()
          accumulate(f - NBUF)
        fire(f)
        return 0
      lax.fori_loop(qf, nf, fire_body, 0)
      return (wr2, nf)

    wr, qf = lax.fori_loop(0, n_windows, window_body,
                           (jnp.int32(0), jnp.int32(0)))

    rem = wr - qf * K
    @pl.when(rem > 0)
    def _():
      off = lax.bitwise_and(qf * K, jnp.int32(CAP - 1))
      io = lax.iota(jnp.int32, L)
      m = io < jnp.full((L,), rem, jnp.int32)
      kdst[pl.ds(off, L)] = jnp.where(m, kdst[pl.ds(off, L)], dummyv)
      ksrc[pl.ds(off, L)] = jnp.where(m, ksrc[pl.ds(off, L)], zerov)
      @pl.when(qf >= NBUF)
      def _():
        drain_one()
        accumulate(qf - NBUF)
      fire(qf)
    nf_all = qf + (rem > 0).astype(jnp.int32)

    def drain_body(f, _):
      drain_one()
      accumulate(f)
      return 0
    lax.fori_loop(jnp.maximum(nf_all - NBUF, 0), nf_all, drain_body, 0)

    @pl.when(wid < NW - 1)
    def _():
      pltpu.sync_copy(acc.at[pl.ds(0, NODES_T)],
                      out_hbm.at[pl.ds(base, NODES_T)])
    @pl.when(wid == NW - 1)
    def _():
      pltpu.sync_copy(acc.at[pl.ds(0, LAST_T)],
                      out_hbm.at[pl.ds(base, LAST_T)])

  return sc_kernel(feature, src_i32, dst_i32)


def _tc_linear_body(h_ref, wt_ref, b_ref, out_ref):
  out_ref[...] = (
      jnp.dot(h_ref[...], wt_ref[...], preferred_element_type=jnp.float32)
      + b_ref[0:1, :])


def _tc_linear(h, wt, b2d):
  m_blk = 1000
  grid = (h.shape[0] // m_blk,)
  return pl.pallas_call(
      _tc_linear_body,
      grid=grid,
      in_specs=[
          pl.BlockSpec((m_blk, D_C), lambda i: (i, 0)),
          pl.BlockSpec((D_C, D_C), lambda i: (0, 0)),
          pl.BlockSpec((8, D_C), lambda i: (0, 0)),
      ],
      out_specs=pl.BlockSpec((m_blk, D_C), lambda i: (i, 0)),
      out_shape=jax.ShapeDtypeStruct((h.shape[0], D_C), jnp.float32),
  )(h, wt, b2d)


@jax.jit
def kernel(feature, edge_index, W, b):
  src = edge_index[0].astype(jnp.int32)
  dst = edge_index[1].astype(jnp.int32)
  n_edges = src.shape[0]
  n_windows = -(-n_edges // WINDOW)
  e_pad = n_windows * WINDOW
  if e_pad != n_edges:
    pad = e_pad - n_edges
    src = jnp.concatenate([src, jnp.zeros((pad,), jnp.int32)])
    dst = jnp.concatenate([dst, jnp.full((pad,), N_NODES_C, jnp.int32)])
  h = _sc_segment_sum(feature, src, dst, n_windows)
  return _tc_linear(h, W.T, jnp.tile(b.reshape(1, D_C), (8, 1)))
